# Initial kernel scaffold; baseline (speedup 1.0000x reference)
import functools

import jax
import jax.numpy as jnp
from jax import lax
from jax.experimental import pallas as pl
from jax.experimental.pallas import tpu as pltpu
from jax.experimental.pallas import tpu_sc as plsc

N = 10000
E = 320000
F = 128

NC = 2
NS = 16
NW = NC * NS
CHUNK = 128
NCHUNK = E // CHUNK
CPW = -(-NCHUNK // NW)
ROWS_PER_SUB = N // NS

_mesh = plsc.VectorSubcoreMesh(core_axis_name="c", subcore_axis_name="s")


@functools.partial(
    pl.kernel,
    mesh=_mesh,
    out_type=jax.ShapeDtypeStruct((NC, N, 16), jnp.float32),
    scratch_types=[
        pltpu.VMEM((CHUNK,), jnp.int32),
        pltpu.VMEM((CHUNK, 16), jnp.float32),
        pltpu.VMEM_SHARED((N, 16), jnp.float32),
    ],
)
def _deg_kernel(dst_hbm, ones_hbm, zeros_hbm, out_hbm, idx_v, ones_v, deg_sh):
    cid = lax.axis_index("c")
    sid = lax.axis_index("s")
    pltpu.sync_copy(ones_hbm, ones_v)
    r0 = sid * ROWS_PER_SUB
    pltpu.sync_copy(zeros_hbm.at[pl.ds(r0, ROWS_PER_SUB)],
                    deg_sh.at[pl.ds(r0, ROWS_PER_SUB)])
    plsc.subcore_barrier()

    w = sid * NC + cid

    @pl.loop(0, CPW)
    def _(i):
        c = w + i * NW

        @pl.when(c < NCHUNK)
        def _():
            base = c * CHUNK
            pltpu.sync_copy(dst_hbm.at[pl.ds(base, CHUNK)], idx_v)
            pltpu.sync_copy(ones_v, deg_sh.at[idx_v], add=True)

    plsc.subcore_barrier()
    pltpu.sync_copy(deg_sh.at[pl.ds(r0, ROWS_PER_SUB)],
                    out_hbm.at[cid].at[pl.ds(r0, ROWS_PER_SUB)])


@functools.partial(
    pl.kernel,
    mesh=_mesh,
    out_type=jax.ShapeDtypeStruct((NC, N, F), jnp.float32),
    scratch_types=[
        pltpu.VMEM((CHUNK,), jnp.int32),
        pltpu.VMEM((CHUNK,), jnp.int32),
        pltpu.VMEM((CHUNK, F), jnp.float32),
        pltpu.VMEM_SHARED((N, F), jnp.float32),
        pltpu.SemaphoreType.DMA,
    ],
)
def _agg_kernel(y_hbm, src_hbm, dst_hbm, zeros_hbm, out_hbm,
                si_v, di_v, rows_v, acc_sh, sem):
    cid = lax.axis_index("c")
    sid = lax.axis_index("s")
    r0 = sid * ROWS_PER_SUB
    pltpu.sync_copy(zeros_hbm.at[pl.ds(r0, ROWS_PER_SUB)],
                    acc_sh.at[pl.ds(r0, ROWS_PER_SUB)])
    plsc.subcore_barrier()

    w = sid * NC + cid

    @pl.loop(0, CPW)
    def _(i):
        c = w + i * NW

        @pl.when(c < NCHUNK)
        def _():
            base = c * CHUNK
            pltpu.sync_copy(src_hbm.at[pl.ds(base, CHUNK)], si_v)
            pltpu.async_copy(y_hbm.at[si_v], rows_v, sem).wait()
            pltpu.sync_copy(dst_hbm.at[pl.ds(base, CHUNK)], di_v)
            pltpu.sync_copy(rows_v, acc_sh.at[di_v], add=True)

    plsc.subcore_barrier()
    pltpu.sync_copy(acc_sh.at[pl.ds(r0, ROWS_PER_SUB)],
                    out_hbm.at[cid].at[pl.ds(r0, ROWS_PER_SUB)])


_MM_BLK = 1000


def _mm_body(x_ref, w_ref, b_ref, h_ref):
    h_ref[...] = (
        jnp.dot(x_ref[...], w_ref[...], preferred_element_type=jnp.float32)
        + b_ref[...]
    )


def _matmul(x, W, b2):
    return pl.pallas_call(
        _mm_body,
        grid=(N // _MM_BLK,),
        in_specs=[
            pl.BlockSpec((_MM_BLK, F), lambda i: (i, 0)),
            pl.BlockSpec((F, F), lambda i: (0, 0)),
            pl.BlockSpec((1, F), lambda i: (0, 0)),
        ],
        out_specs=pl.BlockSpec((_MM_BLK, F), lambda i: (i, 0)),
        out_shape=jax.ShapeDtypeStruct((N, F), jnp.float32),
    )(x, W, b2)


def _dis_from_parts(dp):
    deg = dp[0, :, 0:1] + dp[1, :, 0:1]
    return lax.rsqrt(jnp.maximum(deg, 1.0))


def _scale_body(h_ref, dp_ref, y_ref):
    y_ref[...] = h_ref[...] * _dis_from_parts(dp_ref[...])


def _scale(h, deg_parts):
    return pl.pallas_call(
        _scale_body,
        grid=(N // _MM_BLK,),
        in_specs=[
            pl.BlockSpec((_MM_BLK, F), lambda i: (i, 0)),
            pl.BlockSpec((NC, _MM_BLK, 16), lambda i: (0, i, 0)),
        ],
        out_specs=pl.BlockSpec((_MM_BLK, F), lambda i: (i, 0)),
        out_shape=jax.ShapeDtypeStruct((N, F), jnp.float32),
    )(h, deg_parts)


def _final_body(acc_ref, dp_ref, o_ref):
    z = (acc_ref[0] + acc_ref[1]) * _dis_from_parts(dp_ref[...])
    m = jnp.max(z, axis=1, keepdims=True)
    lse = jnp.log(jnp.sum(jnp.exp(z - m), axis=1, keepdims=True)) + m
    o_ref[...] = z - lse


def _final(acc, deg_parts):
    return pl.pallas_call(
        _final_body,
        grid=(N // _MM_BLK,),
        in_specs=[
            pl.BlockSpec((NC, _MM_BLK, F), lambda i: (0, i, 0)),
            pl.BlockSpec((NC, _MM_BLK, 16), lambda i: (0, i, 0)),
        ],
        out_specs=pl.BlockSpec((_MM_BLK, F), lambda i: (i, 0)),
        out_shape=jax.ShapeDtypeStruct((N, F), jnp.float32),
    )(acc, deg_parts)


def kernel(inputs, edge_index, epoch, W, b):
    del epoch
    src = edge_index[0].astype(jnp.int32)
    dst = edge_index[1].astype(jnp.int32)
    ones16 = jnp.ones((CHUNK, 16), jnp.float32)
    zeros16 = jnp.zeros((N, 16), jnp.float32)
    zerosF = jnp.zeros((N, F), jnp.float32)
    b2 = b.reshape(1, F)

    deg_parts = _deg_kernel(dst, ones16, zeros16)
    h = _matmul(inputs, W, b2)
    y = _scale(h, deg_parts)
    acc = _agg_kernel(y, src, dst, zerosF)
    return _final(acc, deg_parts)


# speedup vs baseline: 19.1275x; 19.1275x over previous
import dataclasses
import functools

import jax
import jax.numpy as jnp
from jax import lax
from jax.experimental import pallas as pl
from jax.experimental.pallas import tpu as pltpu
from jax.experimental.pallas import tpu_sc as plsc

N = 10000
NPAD = 10240
E = 320000
F = 128

NC = 2
NS = 16
NW = NC * NS
CHUNK = 128
NCHUNK = E // CHUNK
CPW = -(-NCHUNK // NW)
ROWS_PER_SUB = NPAD // NS

_mesh = plsc.VectorSubcoreMesh(core_axis_name="c", subcore_axis_name="s")


_cp = pltpu.CompilerParams()
if "needs_layout_passes" in pltpu.CompilerParams.__dataclass_fields__:
    _cp = dataclasses.replace(_cp, needs_layout_passes=False)


@functools.partial(
    pl.kernel,
    mesh=_mesh,
    out_type=jax.ShapeDtypeStruct((NW, NPAD), jnp.float32),
    compiler_params=_cp,
    scratch_types=[
        pltpu.VMEM((CHUNK,), jnp.int32),
        pltpu.VMEM((NPAD,), jnp.float32),
    ],
)
def _deg_kernel(dst_hbm, zeros_hbm, out_hbm, idx_v, deg_v):
    cid = lax.axis_index("c")
    sid = lax.axis_index("s")
    w = sid * NC + cid
    pltpu.sync_copy(zeros_hbm, deg_v)
    ones = jnp.ones((16,), jnp.float32)

    @pl.loop(0, CPW)
    def _(i):
        c = w + i * NW

        @pl.when(c < NCHUNK)
        def _():
            base = c * CHUNK
            pltpu.sync_copy(dst_hbm.at[pl.ds(base, CHUNK)], idx_v)

            @pl.loop(0, CHUNK, step=16)
            def _(j):
                plsc.addupdate_scatter(deg_v, [idx_v[pl.ds(j, 16)]], ones)

    pltpu.sync_copy(deg_v, out_hbm.at[w])


@functools.partial(
    pl.kernel,
    mesh=_mesh,
    out_type=jax.ShapeDtypeStruct((NC, NPAD, F), jnp.float32),
    scratch_types=[
        pltpu.VMEM((CHUNK,), jnp.int32),
        pltpu.VMEM((CHUNK,), jnp.int32),
        pltpu.VMEM((CHUNK, F), jnp.float32),
        pltpu.VMEM_SHARED((NPAD, F), jnp.float32),
        pltpu.SemaphoreType.DMA,
    ],
)
def _agg_kernel(y_hbm, src_hbm, dst_hbm, zeros_hbm, out_hbm,
                si_v, di_v, rows_v, acc_sh, sem):
    cid = lax.axis_index("c")
    sid = lax.axis_index("s")
    r0 = sid * ROWS_PER_SUB
    pltpu.sync_copy(zeros_hbm.at[pl.ds(r0, ROWS_PER_SUB)],
                    acc_sh.at[pl.ds(r0, ROWS_PER_SUB)])
    plsc.subcore_barrier()

    w = sid * NC + cid

    @pl.loop(0, CPW)
    def _(i):
        c = w + i * NW

        @pl.when(c < NCHUNK)
        def _():
            base = c * CHUNK
            pltpu.sync_copy(src_hbm.at[pl.ds(base, CHUNK)], si_v)
            pltpu.async_copy(y_hbm.at[si_v], rows_v, sem).wait()
            pltpu.sync_copy(dst_hbm.at[pl.ds(base, CHUNK)], di_v)
            pltpu.sync_copy(rows_v, acc_sh.at[di_v], add=True)

    plsc.subcore_barrier()
    pltpu.sync_copy(acc_sh.at[pl.ds(r0, ROWS_PER_SUB)],
                    out_hbm.at[cid].at[pl.ds(r0, ROWS_PER_SUB)])


_MM_BLK = 1024


def _mm_body(x_ref, w_ref, b_ref, h_ref):
    h_ref[...] = (
        jnp.dot(x_ref[...], w_ref[...], preferred_element_type=jnp.float32)
        + b_ref[...]
    )


def _matmul(x, W, b2):
    return pl.pallas_call(
        _mm_body,
        grid=(NPAD // _MM_BLK,),
        in_specs=[
            pl.BlockSpec((_MM_BLK, F), lambda i: (i, 0)),
            pl.BlockSpec((F, F), lambda i: (0, 0)),
            pl.BlockSpec((1, F), lambda i: (0, 0)),
        ],
        out_specs=pl.BlockSpec((_MM_BLK, F), lambda i: (i, 0)),
        out_shape=jax.ShapeDtypeStruct((NPAD, F), jnp.float32),
    )(x, W, b2)


def _dis_from_parts(dp):
    deg = jnp.sum(dp, axis=0)[:, None]
    return lax.rsqrt(jnp.maximum(deg, 1.0))


def _scale_body(h_ref, dp_ref, y_ref):
    y_ref[...] = h_ref[...] * _dis_from_parts(dp_ref[...])


def _scale(h, deg_parts):
    return pl.pallas_call(
        _scale_body,
        grid=(NPAD // _MM_BLK,),
        in_specs=[
            pl.BlockSpec((_MM_BLK, F), lambda i: (i, 0)),
            pl.BlockSpec((NW, _MM_BLK), lambda i: (0, i)),
        ],
        out_specs=pl.BlockSpec((_MM_BLK, F), lambda i: (i, 0)),
        out_shape=jax.ShapeDtypeStruct((NPAD, F), jnp.float32),
    )(h, deg_parts)


def _final_body(acc_ref, dp_ref, o_ref):
    z = (acc_ref[0] + acc_ref[1]) * _dis_from_parts(dp_ref[...])
    m = jnp.max(z, axis=1, keepdims=True)
    lse = jnp.log(jnp.sum(jnp.exp(z - m), axis=1, keepdims=True)) + m
    o_ref[...] = z - lse


def _final(acc, deg_parts):
    return pl.pallas_call(
        _final_body,
        grid=(NPAD // _MM_BLK,),
        in_specs=[
            pl.BlockSpec((NC, _MM_BLK, F), lambda i: (0, i, 0)),
            pl.BlockSpec((NW, _MM_BLK), lambda i: (0, i)),
        ],
        out_specs=pl.BlockSpec((_MM_BLK, F), lambda i: (i, 0)),
        out_shape=jax.ShapeDtypeStruct((NPAD, F), jnp.float32),
    )(acc, deg_parts)


def kernel(inputs, edge_index, epoch, W, b):
    del epoch
    src = edge_index[0].astype(jnp.int32)
    dst = edge_index[1].astype(jnp.int32)
    zeros1 = jnp.zeros((NPAD,), jnp.float32)
    zerosF = jnp.zeros((NPAD, F), jnp.float32)
    b2 = b.reshape(1, F)
    x_pad = jnp.pad(inputs, ((0, NPAD - N), (0, 0)))

    deg_parts = _deg_kernel(dst, zeros1)
    h = _matmul(x_pad, W, b2)
    y = _scale(h, deg_parts)
    acc = _agg_kernel(y, src, dst, zerosF)
    return _final(acc, deg_parts)[:N]


# speedup vs baseline: 29.8492x; 1.5605x over previous
import dataclasses
import functools

import jax
import jax.numpy as jnp
from jax import lax
from jax.experimental import pallas as pl
from jax.experimental.pallas import tpu as pltpu
from jax.experimental.pallas import tpu_sc as plsc

N = 10000
NPAD = 10240
E = 320000
F = 128

NC = 2
NS = 16
NW = NC * NS
CHUNK = 128
NCHUNK = E // CHUNK
CPW = -(-NCHUNK // NW)
ROWS_PER_SUB = NPAD // NS
EPW = E // NW
DEGBLK = 1024

_mesh = plsc.VectorSubcoreMesh(core_axis_name="c", subcore_axis_name="s")


_cp = pltpu.CompilerParams()
if "needs_layout_passes" in pltpu.CompilerParams.__dataclass_fields__:
    _cp = dataclasses.replace(_cp, needs_layout_passes=False)


@functools.partial(
    pl.kernel,
    mesh=_mesh,
    out_type=jax.ShapeDtypeStruct((NW, NPAD), jnp.float32),
    compiler_params=_cp,
    scratch_types=[
        pltpu.VMEM((DEGBLK,), jnp.int32),
        pltpu.VMEM((NPAD,), jnp.float32),
    ],
)
def _deg_kernel(dst_hbm, zeros_hbm, out_hbm, idx_v, deg_v):
    cid = lax.axis_index("c")
    sid = lax.axis_index("s")
    w = sid * NC + cid
    pltpu.sync_copy(zeros_hbm, deg_v)
    ones = jnp.ones((16,), jnp.float32)
    base = w * EPW

    def scan_block(off, size):
        pltpu.sync_copy(dst_hbm.at[pl.ds(base + off, size)],
                        idx_v.at[pl.ds(0, size)])

        @pl.loop(0, size, step=16)
        def _(j):
            plsc.addupdate_scatter(deg_v, [idx_v[pl.ds(j, 16)]], ones)

    @pl.loop(0, EPW // DEGBLK)
    def _(k):
        scan_block(k * DEGBLK, DEGBLK)

    if EPW % DEGBLK:
        scan_block((EPW // DEGBLK) * DEGBLK, EPW % DEGBLK)

    pltpu.sync_copy(deg_v, out_hbm.at[w])


@functools.partial(
    pl.kernel,
    mesh=_mesh,
    out_type=jax.ShapeDtypeStruct((NC, NPAD, F), jnp.float32),
    scratch_types=[
        pltpu.VMEM((CHUNK,), jnp.int32),
        pltpu.VMEM((CHUNK,), jnp.int32),
        pltpu.VMEM((CHUNK,), jnp.int32),
        pltpu.VMEM((CHUNK,), jnp.int32),
        pltpu.VMEM((CHUNK, F), jnp.float32),
        pltpu.VMEM((CHUNK, F), jnp.float32),
        pltpu.VMEM_SHARED((NPAD, F), jnp.float32),
        pltpu.SemaphoreType.DMA,
        pltpu.SemaphoreType.DMA,
    ],
)
def _agg_kernel(y_hbm, src_hbm, dst_hbm, zeros_hbm, out_hbm,
                si_a, si_b, di_a, di_b, rows_a, rows_b, acc_sh, sem_a, sem_b):
    cid = lax.axis_index("c")
    sid = lax.axis_index("s")
    r0 = sid * ROWS_PER_SUB
    pltpu.sync_copy(zeros_hbm.at[pl.ds(r0, ROWS_PER_SUB)],
                    acc_sh.at[pl.ds(r0, ROWS_PER_SUB)])
    plsc.subcore_barrier()

    w = sid * NC + cid

    def start(c, si, di, rows, sem):
        base = c * CHUNK
        pltpu.sync_copy(src_hbm.at[pl.ds(base, CHUNK)], si)
        pltpu.sync_copy(dst_hbm.at[pl.ds(base, CHUNK)], di)
        pltpu.make_async_copy(y_hbm.at[si], rows, sem).start()

    def finish(di, rows, sem):
        pltpu.make_async_copy(y_hbm.at[pl.ds(0, CHUNK)], rows, sem).wait()
        pltpu.sync_copy(rows, acc_sh.at[di], add=True)

    start(w, si_a, di_a, rows_a, sem_a)

    @pl.loop(0, (CPW + 1) // 2)
    def _(j):
        c0 = w + (2 * j) * NW
        c1 = c0 + NW
        c2 = c1 + NW

        @pl.when(c1 < NCHUNK)
        def _():
            start(c1, si_b, di_b, rows_b, sem_b)

        @pl.when(c0 < NCHUNK)
        def _():
            finish(di_a, rows_a, sem_a)

        @pl.when(c2 < NCHUNK)
        def _():
            start(c2, si_a, di_a, rows_a, sem_a)

        @pl.when(c1 < NCHUNK)
        def _():
            finish(di_b, rows_b, sem_b)

    plsc.subcore_barrier()
    pltpu.sync_copy(acc_sh.at[pl.ds(r0, ROWS_PER_SUB)],
                    out_hbm.at[cid].at[pl.ds(r0, ROWS_PER_SUB)])


_MM_BLK = 1024


def _mm_body(x_ref, w_ref, b_ref, h_ref):
    h_ref[...] = (
        jnp.dot(x_ref[...], w_ref[...], preferred_element_type=jnp.float32)
        + b_ref[...]
    )


def _matmul(x, W, b2):
    return pl.pallas_call(
        _mm_body,
        grid=(NPAD // _MM_BLK,),
        in_specs=[
            pl.BlockSpec((_MM_BLK, F), lambda i: (i, 0)),
            pl.BlockSpec((F, F), lambda i: (0, 0)),
            pl.BlockSpec((1, F), lambda i: (0, 0)),
        ],
        out_specs=pl.BlockSpec((_MM_BLK, F), lambda i: (i, 0)),
        out_shape=jax.ShapeDtypeStruct((NPAD, F), jnp.float32),
    )(x, W, b2)


def _dis_from_parts(dp):
    deg = jnp.sum(dp, axis=0)[:, None]
    return lax.rsqrt(jnp.maximum(deg, 1.0))


def _scale_body(h_ref, dp_ref, y_ref):
    y_ref[...] = h_ref[...] * _dis_from_parts(dp_ref[...])


def _scale(h, deg_parts):
    return pl.pallas_call(
        _scale_body,
        grid=(NPAD // _MM_BLK,),
        in_specs=[
            pl.BlockSpec((_MM_BLK, F), lambda i: (i, 0)),
            pl.BlockSpec((NW, _MM_BLK), lambda i: (0, i)),
        ],
        out_specs=pl.BlockSpec((_MM_BLK, F), lambda i: (i, 0)),
        out_shape=jax.ShapeDtypeStruct((NPAD, F), jnp.float32),
    )(h, deg_parts)


def _final_body(acc_ref, dp_ref, o_ref):
    z = (acc_ref[0] + acc_ref[1]) * _dis_from_parts(dp_ref[...])
    m = jnp.max(z, axis=1, keepdims=True)
    lse = jnp.log(jnp.sum(jnp.exp(z - m), axis=1, keepdims=True)) + m
    o_ref[...] = z - lse


def _final(acc, deg_parts):
    return pl.pallas_call(
        _final_body,
        grid=(NPAD // _MM_BLK,),
        in_specs=[
            pl.BlockSpec((NC, _MM_BLK, F), lambda i: (0, i, 0)),
            pl.BlockSpec((NW, _MM_BLK), lambda i: (0, i)),
        ],
        out_specs=pl.BlockSpec((_MM_BLK, F), lambda i: (i, 0)),
        out_shape=jax.ShapeDtypeStruct((NPAD, F), jnp.float32),
    )(acc, deg_parts)


def kernel(inputs, edge_index, epoch, W, b):
    del epoch
    src = edge_index[0].astype(jnp.int32)
    dst = edge_index[1].astype(jnp.int32)
    zeros1 = jnp.zeros((NPAD,), jnp.float32)
    zerosF = jnp.zeros((NPAD, F), jnp.float32)
    b2 = b.reshape(1, F)
    x_pad = jnp.pad(inputs, ((0, NPAD - N), (0, 0)))

    deg_parts = _deg_kernel(dst, zeros1)
    h = _matmul(x_pad, W, b2)
    y = _scale(h, deg_parts)
    acc = _agg_kernel(y, src, dst, zerosF)
    return _final(acc, deg_parts)[:N]
